# baseline (speedup 1.0000x reference)
import functools
import math

import jax
import jax.numpy as jnp
from jax.experimental import pallas as pl
from jax.experimental.pallas import tpu as pltpu

B = 16
C = 3
H = 224
P = 16
E = 768
NH = 12
NC = 1000
NEXP = 6
TOPK = 2
HID = 3072
NPATCH = (H // P) ** 2
T = B * NPATCH

_BT = 448
_SQRT2 = math.sqrt(2.0)

_BA = 256
_NB = (T * TOPK + NEXP * (_BA - 1) + _BA - 1) // _BA
_PADT = _NB * _BA
_TPAD = T + 8


def _moe_ffn_kernel(be_ref, x_ref, w1_ref, b1_ref, w2_ref, b2_ref,
                    g_ref, out_ref):
    e = be_ref[pl.program_id(0)]
    x = x_ref[...]
    w1 = w1_ref[pl.ds(e, 1)][0]
    h = jax.lax.dot_general(x, w1, (((1,), (1,)), ((), ())),
                            preferred_element_type=jnp.float32)
    h = h + b1_ref[pl.ds(e, 1)][0, 0]
    h = 0.5 * h * (1.0 + jax.lax.erf(h / _SQRT2))
    w2 = w2_ref[pl.ds(e, 1)][0]
    eo = jax.lax.dot_general(h.astype(jnp.bfloat16), w2,
                             (((1,), (1,)), ((), ())),
                             preferred_element_type=jnp.float32)
    eo = eo + b2_ref[pl.ds(e, 1)][0, 0]
    out_ref[...] = eo * g_ref[...]


def _moe_sparse(flat, flat_i, flat_p, exp_w1, exp_b1, exp_w2, exp_b2):
    ids = flat_i.reshape(-1)
    oneh = (ids[:, None] == jnp.arange(NEXP, dtype=ids.dtype)[None, :])
    incl = jnp.cumsum(oneh.astype(jnp.int32), axis=0)
    rank = jnp.take_along_axis(incl, ids[:, None], axis=1)[:, 0] - 1
    counts = incl[-1]
    padded = ((counts + _BA - 1) // _BA) * _BA
    off = jnp.concatenate([jnp.zeros((1,), jnp.int32),
                           jnp.cumsum(padded).astype(jnp.int32)])
    pos = off[ids] + rank
    tok = (jnp.arange(T * TOPK, dtype=jnp.int32) // TOPK)
    row_token = jnp.full((_PADT,), T, jnp.int32).at[pos].set(tok)
    row_gate = jnp.zeros((_PADT,), jnp.float32).at[pos].set(flat_p.reshape(-1))
    starts = jnp.arange(_NB, dtype=jnp.int32) * _BA
    block_expert = jnp.minimum(
        jnp.searchsorted(off[1:], starts, side='right'),
        NEXP - 1).astype(jnp.int32)

    z_pad = jnp.concatenate(
        [flat, jnp.zeros((_TPAD - T, E), jnp.float32)], axis=0)
    x_sorted = jnp.take(z_pad, row_token, axis=0).astype(jnp.bfloat16)

    grid_spec = pltpu.PrefetchScalarGridSpec(
        num_scalar_prefetch=1,
        grid=(_NB,),
        in_specs=[
            pl.BlockSpec((_BA, E), lambda i, be: (i, 0)),
            pl.BlockSpec((NEXP, HID, E), lambda i, be: (0, 0, 0)),
            pl.BlockSpec((NEXP, 1, HID), lambda i, be: (0, 0, 0)),
            pl.BlockSpec((NEXP, E, HID), lambda i, be: (0, 0, 0)),
            pl.BlockSpec((NEXP, 1, E), lambda i, be: (0, 0, 0)),
            pl.BlockSpec((_BA, 1), lambda i, be: (i, 0)),
        ],
        out_specs=pl.BlockSpec((_BA, E), lambda i, be: (i, 0)),
    )
    buf = pl.pallas_call(
        _moe_ffn_kernel,
        grid_spec=grid_spec,
        out_shape=jax.ShapeDtypeStruct((_PADT, E), jnp.float32),
    )(block_expert, x_sorted, exp_w1.astype(jnp.bfloat16),
      exp_b1.reshape(NEXP, 1, HID), exp_w2.astype(jnp.bfloat16),
      exp_b2.reshape(NEXP, 1, E), row_gate[:, None])
    pos2 = pos.reshape(T, TOPK)
    return jnp.take(buf, pos2[:, 0], axis=0) + jnp.take(buf, pos2[:, 1], axis=0)


def _moe_dense_kernel(z_ref, w1_ref, b1_ref, w2_ref, b2_ref, gates_ref, out_ref):
    e = pl.program_id(1)
    z = z_ref[...]
    h = jax.lax.dot_general(z, w1_ref[0], (((1,), (1,)), ((), ())),
                            preferred_element_type=jnp.float32)
    h = h + b1_ref[0, 0]
    h = 0.5 * h * (1.0 + jax.lax.erf(h / _SQRT2))
    eo = jax.lax.dot_general(h, w2_ref[0], (((1,), (1,)), ((), ())),
                             preferred_element_type=jnp.float32)
    eo = eo + b2_ref[0, 0]
    lane = jax.lax.broadcasted_iota(jnp.int32, (_BT, NEXP), 1)
    g = jnp.sum(jnp.where(lane == e, gates_ref[...], 0.0), axis=1, keepdims=True)
    contrib = eo * g

    @pl.when(e == 0)
    def _init():
        out_ref[...] = contrib

    @pl.when(e != 0)
    def _acc():
        out_ref[...] += contrib


def _moe_dense(flat, gates, exp_w1, exp_b1, exp_w2, exp_b2):
    grid = (T // _BT, NEXP)
    return pl.pallas_call(
        _moe_dense_kernel,
        grid=grid,
        in_specs=[
            pl.BlockSpec((_BT, E), lambda t, e: (t, 0)),
            pl.BlockSpec((1, HID, E), lambda t, e: (e, 0, 0)),
            pl.BlockSpec((1, 1, HID), lambda t, e: (e, 0, 0)),
            pl.BlockSpec((1, E, HID), lambda t, e: (e, 0, 0)),
            pl.BlockSpec((1, 1, E), lambda t, e: (e, 0, 0)),
            pl.BlockSpec((_BT, NEXP), lambda t, e: (t, 0)),
        ],
        out_specs=pl.BlockSpec((_BT, E), lambda t, e: (t, 0)),
        out_shape=jax.ShapeDtypeStruct((T, E), jnp.float32),
    )(flat, exp_w1, exp_b1.reshape(NEXP, 1, HID), exp_w2,
      exp_b2.reshape(NEXP, 1, E), gates)


_NPAD = 224
_DH = E // NH


def _patch_kernel(p_ref, w_ref, b_ref, pos_ref, o_ref):
    p = p_ref[0].astype(jnp.bfloat16)
    zb = jax.lax.dot_general(p, w_ref[...], (((1,), (1,)), ((), ())),
                             preferred_element_type=jnp.float32)
    o_ref[0] = zb + b_ref[0] + pos_ref[0]


def _patch_embed(patches, patch_w, patch_b, pos_embed):
    return pl.pallas_call(
        _patch_kernel,
        grid=(B,),
        in_specs=[
            pl.BlockSpec((1, NPATCH, C * P * P), lambda b: (b, 0, 0)),
            pl.BlockSpec((E, C * P * P), lambda b: (0, 0)),
            pl.BlockSpec((1, E), lambda b: (0, 0)),
            pl.BlockSpec((1, NPATCH, E), lambda b: (0, 0, 0)),
        ],
        out_specs=pl.BlockSpec((1, NPATCH, E), lambda b: (b, 0, 0)),
        out_shape=jax.ShapeDtypeStruct((B, NPATCH, E), jnp.float32),
    )(patches, patch_w.reshape(E, C * P * P).astype(jnp.bfloat16),
      patch_b[None, :], pos_embed)


def _attn_kernel(z_ref, ln_g_ref, ln_b_ref, win_ref, bin_ref, wout_ref,
                 bout_ref, out_ref):
    z = z_ref[0]
    m = jnp.mean(z, axis=-1, keepdims=True)
    v = jnp.mean((z - m) ** 2, axis=-1, keepdims=True)
    zn = (z - m) / jnp.sqrt(v + 1e-5) * ln_g_ref[0] + ln_b_ref[0]
    qkv = jax.lax.dot_general(zn.astype(jnp.bfloat16), win_ref[...],
                              (((1,), (1,)), ((), ())),
                              preferred_element_type=jnp.float32)
    qkv = qkv + bin_ref[0]
    kmask = (jax.lax.broadcasted_iota(jnp.int32, (_NPAD, _NPAD), 1)
             >= NPATCH)
    heads = []
    for h in range(NH):
        qh = qkv[:, h * _DH:(h + 1) * _DH]
        kh = qkv[:, E + h * _DH:E + (h + 1) * _DH]
        vh = qkv[:, 2 * E + h * _DH:2 * E + (h + 1) * _DH]
        s = jax.lax.dot_general(qh, kh, (((1,), (1,)), ((), ())),
                                preferred_element_type=jnp.float32)
        s = s * (1.0 / math.sqrt(_DH))
        s = jnp.where(kmask, -1e30, s)
        s = s - jnp.max(s, axis=-1, keepdims=True)
        p = jnp.exp(s)
        p = p / jnp.sum(p, axis=-1, keepdims=True)
        oh = jax.lax.dot_general(p.astype(jnp.bfloat16),
                                 vh.astype(jnp.bfloat16),
                                 (((1,), (0,)), ((), ())),
                                 preferred_element_type=jnp.float32)
        heads.append(oh)
    ao = jnp.concatenate(heads, axis=1)
    ao = jax.lax.dot_general(ao.astype(jnp.bfloat16), wout_ref[...],
                             (((1,), (1,)), ((), ())),
                             preferred_element_type=jnp.float32)
    out_ref[0] = z + ao + bout_ref[0]


def _attention(z, ln1_g, ln1_b, attn_in_w, attn_in_b, attn_out_w, attn_out_b):
    zp = jnp.pad(z, ((0, 0), (0, _NPAD - NPATCH), (0, 0)))
    out = pl.pallas_call(
        _attn_kernel,
        grid=(B,),
        in_specs=[
            pl.BlockSpec((1, _NPAD, E), lambda b: (b, 0, 0)),
            pl.BlockSpec((1, E), lambda b: (0, 0)),
            pl.BlockSpec((1, E), lambda b: (0, 0)),
            pl.BlockSpec((3 * E, E), lambda b: (0, 0)),
            pl.BlockSpec((1, 3 * E), lambda b: (0, 0)),
            pl.BlockSpec((E, E), lambda b: (0, 0)),
            pl.BlockSpec((1, E), lambda b: (0, 0)),
        ],
        out_specs=pl.BlockSpec((1, _NPAD, E), lambda b: (b, 0, 0)),
        out_shape=jax.ShapeDtypeStruct((B, _NPAD, E), jnp.float32),
    )(zp, ln1_g[None, :], ln1_b[None, :], attn_in_w.astype(jnp.bfloat16),
      attn_in_b[None, :], attn_out_w.astype(jnp.bfloat16), attn_out_b[None, :])
    return out[:, :NPATCH, :]


def _layernorm(x, g, b):
    m = jnp.mean(x, axis=-1, keepdims=True)
    v = jnp.mean((x - m) ** 2, axis=-1, keepdims=True)
    return (x - m) / jnp.sqrt(v + 1e-5) * g + b


def kernel(x, patch_w, patch_b, pos_embed, ln1_g, ln1_b, attn_in_w, attn_in_b,
           attn_out_w, attn_out_b, router_w, router_b, exp_w1, exp_b1, exp_w2,
           exp_b2, ln2_g, ln2_b, head_w, head_b):
    Bn = x.shape[0]
    hp = H // P
    patches = x.reshape(Bn, C, hp, P, hp, P).transpose(0, 2, 4, 1, 3, 5)
    patches = patches.reshape(Bn, hp * hp, C * P * P)
    z = _patch_embed(patches, patch_w, patch_b, pos_embed)

    z = _attention(z, ln1_g, ln1_b, attn_in_w, attn_in_b,
                   attn_out_w, attn_out_b)

    logits = z @ router_w.T + router_b
    probs = jax.nn.softmax(logits, axis=-1)
    topk_p, topk_i = jax.lax.top_k(probs, TOPK)
    flat = z.reshape(T, E)
    flat_i = topk_i.reshape(T, TOPK)
    flat_p = topk_p.reshape(T, TOPK)

    out = _moe_sparse(flat, flat_i, flat_p, exp_w1, exp_b1, exp_w2, exp_b2)

    z = out.reshape(Bn, -1, E)
    z = _layernorm(z, ln2_g, ln2_b)
    pooled = jnp.mean(z, axis=1)
    return pooled @ head_w.T + head_b


# speedup vs baseline: 1.0699x; 1.0699x over previous
import functools
import math

import jax
import jax.numpy as jnp
from jax.experimental import pallas as pl
from jax.experimental.pallas import tpu as pltpu

B = 16
C = 3
H = 224
P = 16
E = 768
NH = 12
NC = 1000
NEXP = 6
TOPK = 2
HID = 3072
NPATCH = (H // P) ** 2
T = B * NPATCH

_BT = 448
_SQRT2 = math.sqrt(2.0)

_BA = 256
_NB = (T * TOPK + NEXP * (_BA - 1) + _BA - 1) // _BA
_PADT = _NB * _BA
_TPAD = T + 8


def _moe_ffn_kernel(be_ref, x_ref, w1_ref, b1_ref, w2_ref, b2_ref,
                    g_ref, out_ref):
    e = be_ref[pl.program_id(0)]
    x = x_ref[...]
    w1 = w1_ref[pl.ds(e, 1)][0]
    h = jax.lax.dot_general(x, w1, (((1,), (1,)), ((), ())),
                            preferred_element_type=jnp.float32)
    h = h + b1_ref[pl.ds(e, 1)][0, 0]
    h = 0.5 * h * (1.0 + jax.lax.erf(h / _SQRT2))
    w2 = w2_ref[pl.ds(e, 1)][0]
    eo = jax.lax.dot_general(h.astype(jnp.bfloat16), w2,
                             (((1,), (1,)), ((), ())),
                             preferred_element_type=jnp.float32)
    eo = eo + b2_ref[pl.ds(e, 1)][0, 0]
    out_ref[...] = eo * g_ref[...]


def _moe_sparse(flat, flat_i, flat_p, exp_w1, exp_b1, exp_w2, exp_b2):
    ids = flat_i.reshape(-1)
    oneh = (ids[:, None] == jnp.arange(NEXP, dtype=ids.dtype)[None, :])
    incl = jnp.cumsum(oneh.astype(jnp.int32), axis=0)
    rank = jnp.take_along_axis(incl, ids[:, None], axis=1)[:, 0] - 1
    counts = incl[-1]
    padded = ((counts + _BA - 1) // _BA) * _BA
    off = jnp.concatenate([jnp.zeros((1,), jnp.int32),
                           jnp.cumsum(padded).astype(jnp.int32)])
    pos = off[ids] + rank
    tok = (jnp.arange(T * TOPK, dtype=jnp.int32) // TOPK)
    row_token = jnp.full((_PADT,), T, jnp.int32).at[pos].set(tok)
    row_gate = jnp.zeros((_PADT,), jnp.float32).at[pos].set(flat_p.reshape(-1))
    starts = jnp.arange(_NB, dtype=jnp.int32) * _BA
    block_expert = jnp.minimum(
        jnp.searchsorted(off[1:], starts, side='right'),
        NEXP - 1).astype(jnp.int32)

    z_pad = jnp.concatenate(
        [flat, jnp.zeros((_TPAD - T, E), jnp.float32)], axis=0)
    x_sorted = jnp.take(z_pad, row_token, axis=0).astype(jnp.bfloat16)

    grid_spec = pltpu.PrefetchScalarGridSpec(
        num_scalar_prefetch=1,
        grid=(_NB,),
        in_specs=[
            pl.BlockSpec((_BA, E), lambda i, be: (i, 0)),
            pl.BlockSpec((NEXP, HID, E), lambda i, be: (0, 0, 0)),
            pl.BlockSpec((NEXP, 1, HID), lambda i, be: (0, 0, 0)),
            pl.BlockSpec((NEXP, E, HID), lambda i, be: (0, 0, 0)),
            pl.BlockSpec((NEXP, 1, E), lambda i, be: (0, 0, 0)),
            pl.BlockSpec((_BA, 1), lambda i, be: (i, 0)),
        ],
        out_specs=pl.BlockSpec((_BA, E), lambda i, be: (i, 0)),
    )
    buf = pl.pallas_call(
        _moe_ffn_kernel,
        grid_spec=grid_spec,
        out_shape=jax.ShapeDtypeStruct((_PADT, E), jnp.float32),
    )(block_expert, x_sorted, exp_w1.astype(jnp.bfloat16),
      exp_b1.reshape(NEXP, 1, HID), exp_w2.astype(jnp.bfloat16),
      exp_b2.reshape(NEXP, 1, E), row_gate[:, None])
    pos2 = pos.reshape(T, TOPK)
    return jnp.take(buf, pos2[:, 0], axis=0) + jnp.take(buf, pos2[:, 1], axis=0)


def _moe_dense_kernel(z_ref, w1_ref, b1_ref, w2_ref, b2_ref, gates_ref, out_ref):
    e = pl.program_id(1)
    z = z_ref[...]
    h = jax.lax.dot_general(z, w1_ref[0], (((1,), (1,)), ((), ())),
                            preferred_element_type=jnp.float32)
    h = h + b1_ref[0, 0]
    h = 0.5 * h * (1.0 + jax.lax.erf(h / _SQRT2))
    eo = jax.lax.dot_general(h, w2_ref[0], (((1,), (1,)), ((), ())),
                             preferred_element_type=jnp.float32)
    eo = eo + b2_ref[0, 0]
    lane = jax.lax.broadcasted_iota(jnp.int32, (_BT, NEXP), 1)
    g = jnp.sum(jnp.where(lane == e, gates_ref[...], 0.0), axis=1, keepdims=True)
    contrib = eo * g

    @pl.when(e == 0)
    def _init():
        out_ref[...] = contrib

    @pl.when(e != 0)
    def _acc():
        out_ref[...] += contrib


def _moe_dense(flat, gates, exp_w1, exp_b1, exp_w2, exp_b2):
    grid = (T // _BT, NEXP)
    return pl.pallas_call(
        _moe_dense_kernel,
        grid=grid,
        in_specs=[
            pl.BlockSpec((_BT, E), lambda t, e: (t, 0)),
            pl.BlockSpec((1, HID, E), lambda t, e: (e, 0, 0)),
            pl.BlockSpec((1, 1, HID), lambda t, e: (e, 0, 0)),
            pl.BlockSpec((1, E, HID), lambda t, e: (e, 0, 0)),
            pl.BlockSpec((1, 1, E), lambda t, e: (e, 0, 0)),
            pl.BlockSpec((_BT, NEXP), lambda t, e: (t, 0)),
        ],
        out_specs=pl.BlockSpec((_BT, E), lambda t, e: (t, 0)),
        out_shape=jax.ShapeDtypeStruct((T, E), jnp.float32),
    )(flat, exp_w1, exp_b1.reshape(NEXP, 1, HID), exp_w2,
      exp_b2.reshape(NEXP, 1, E), gates)


_NPAD = 224
_DH = E // NH


def _patch_kernel(p_ref, w_ref, b_ref, pos_ref, o_ref):
    p = p_ref[0].astype(jnp.bfloat16)
    zb = jax.lax.dot_general(p, w_ref[...], (((1,), (1,)), ((), ())),
                             preferred_element_type=jnp.float32)
    o_ref[0] = zb + b_ref[0] + pos_ref[0]


def _patch_embed(patches, patch_w, patch_b, pos_embed):
    return pl.pallas_call(
        _patch_kernel,
        grid=(B,),
        in_specs=[
            pl.BlockSpec((1, NPATCH, C * P * P), lambda b: (b, 0, 0)),
            pl.BlockSpec((E, C * P * P), lambda b: (0, 0)),
            pl.BlockSpec((1, E), lambda b: (0, 0)),
            pl.BlockSpec((1, NPATCH, E), lambda b: (0, 0, 0)),
        ],
        out_specs=pl.BlockSpec((1, NPATCH, E), lambda b: (b, 0, 0)),
        out_shape=jax.ShapeDtypeStruct((B, NPATCH, E), jnp.float32),
    )(patches, patch_w.reshape(E, C * P * P).astype(jnp.bfloat16),
      patch_b[None, :], pos_embed)


def _attn_kernel(z_ref, ln_g_ref, ln_b_ref, win_ref, bin_ref, wout_ref,
                 bout_ref, out_ref):
    z = z_ref[0]
    m = jnp.mean(z, axis=-1, keepdims=True)
    v = jnp.mean((z - m) ** 2, axis=-1, keepdims=True)
    zn = (z - m) / jnp.sqrt(v + 1e-5) * ln_g_ref[0] + ln_b_ref[0]
    qkv = jax.lax.dot_general(zn.astype(jnp.bfloat16), win_ref[...],
                              (((1,), (1,)), ((), ())),
                              preferred_element_type=jnp.float32)
    qkv = qkv + bin_ref[0]
    kmask = (jax.lax.broadcasted_iota(jnp.int32, (_NPAD, _NPAD), 1)
             >= NPATCH)
    heads = []
    for h in range(NH):
        qh = qkv[:, h * _DH:(h + 1) * _DH]
        kh = qkv[:, E + h * _DH:E + (h + 1) * _DH]
        vh = qkv[:, 2 * E + h * _DH:2 * E + (h + 1) * _DH]
        s = jax.lax.dot_general(qh, kh, (((1,), (1,)), ((), ())),
                                preferred_element_type=jnp.float32)
        s = s * (1.0 / math.sqrt(_DH))
        s = jnp.where(kmask, -1e30, s)
        s = s - jnp.max(s, axis=-1, keepdims=True)
        p = jnp.exp(s)
        p = p / jnp.sum(p, axis=-1, keepdims=True)
        oh = jax.lax.dot_general(p.astype(jnp.bfloat16),
                                 vh.astype(jnp.bfloat16),
                                 (((1,), (0,)), ((), ())),
                                 preferred_element_type=jnp.float32)
        heads.append(oh)
    ao = jnp.concatenate(heads, axis=1)
    ao = jax.lax.dot_general(ao.astype(jnp.bfloat16), wout_ref[...],
                             (((1,), (1,)), ((), ())),
                             preferred_element_type=jnp.float32)
    out_ref[0] = z + ao + bout_ref[0]


def _attention(z, ln1_g, ln1_b, attn_in_w, attn_in_b, attn_out_w, attn_out_b):
    zp = jnp.pad(z, ((0, 0), (0, _NPAD - NPATCH), (0, 0)))
    out = pl.pallas_call(
        _attn_kernel,
        grid=(B,),
        in_specs=[
            pl.BlockSpec((1, _NPAD, E), lambda b: (b, 0, 0)),
            pl.BlockSpec((1, E), lambda b: (0, 0)),
            pl.BlockSpec((1, E), lambda b: (0, 0)),
            pl.BlockSpec((3 * E, E), lambda b: (0, 0)),
            pl.BlockSpec((1, 3 * E), lambda b: (0, 0)),
            pl.BlockSpec((E, E), lambda b: (0, 0)),
            pl.BlockSpec((1, E), lambda b: (0, 0)),
        ],
        out_specs=pl.BlockSpec((1, _NPAD, E), lambda b: (b, 0, 0)),
        out_shape=jax.ShapeDtypeStruct((B, _NPAD, E), jnp.float32),
    )(zp, ln1_g[None, :], ln1_b[None, :], attn_in_w.astype(jnp.bfloat16),
      attn_in_b[None, :], attn_out_w.astype(jnp.bfloat16), attn_out_b[None, :])
    return out[:, :NPATCH, :]


def _layernorm(x, g, b):
    m = jnp.mean(x, axis=-1, keepdims=True)
    v = jnp.mean((x - m) ** 2, axis=-1, keepdims=True)
    return (x - m) / jnp.sqrt(v + 1e-5) * g + b


def kernel(x, patch_w, patch_b, pos_embed, ln1_g, ln1_b, attn_in_w, attn_in_b,
           attn_out_w, attn_out_b, router_w, router_b, exp_w1, exp_b1, exp_w2,
           exp_b2, ln2_g, ln2_b, head_w, head_b):
    Bn = x.shape[0]
    hp = H // P
    patches = x.reshape(Bn, C, hp, P, hp, P).transpose(0, 2, 4, 1, 3, 5)
    patches = patches.reshape(Bn, hp * hp, C * P * P)
    z = jax.lax.dot_general(
        patches.astype(jnp.bfloat16),
        patch_w.reshape(E, C * P * P).astype(jnp.bfloat16),
        (((2,), (1,)), ((), ())), preferred_element_type=jnp.float32)
    z = z + patch_b + pos_embed

    z = _attention(z, ln1_g, ln1_b, attn_in_w, attn_in_b,
                   attn_out_w, attn_out_b)

    logits = z @ router_w.T + router_b
    probs = jax.nn.softmax(logits, axis=-1)
    topk_p, topk_i = jax.lax.top_k(probs, TOPK)
    flat = z.reshape(T, E)
    flat_i = topk_i.reshape(T, TOPK)
    flat_p = topk_p.reshape(T, TOPK)

    out = _moe_sparse(flat, flat_i, flat_p, exp_w1, exp_b1, exp_w2, exp_b2)

    z = out.reshape(Bn, -1, E)
    z = _layernorm(z, ln2_g, ln2_b)
    pooled = jnp.mean(z, axis=1)
    return pooled @ head_w.T + head_b


# speedup vs baseline: 1.0837x; 1.0129x over previous
import functools
import math

import jax
import jax.numpy as jnp
from jax.experimental import pallas as pl
from jax.experimental.pallas import tpu as pltpu

B = 16
C = 3
H = 224
P = 16
E = 768
NH = 12
NC = 1000
NEXP = 6
TOPK = 2
HID = 3072
NPATCH = (H // P) ** 2
T = B * NPATCH

_SQRT2 = math.sqrt(2.0)

_BA = 256
_NB = (T * TOPK + NEXP * (_BA - 1) + _BA - 1) // _BA
_PADT = _NB * _BA
_TPAD = T + 8


def _moe_ffn_kernel(be_ref, x_ref, w1_ref, b1_ref, w2_ref, b2_ref,
                    g_ref, out_ref):
    e = be_ref[pl.program_id(0)]
    x = x_ref[...]
    w1 = w1_ref[pl.ds(e, 1)][0]
    h = jax.lax.dot_general(x, w1, (((1,), (1,)), ((), ())),
                            preferred_element_type=jnp.float32)
    h = h + b1_ref[pl.ds(e, 1)][0, 0]
    h = 0.5 * h * (1.0 + jax.lax.erf(h / _SQRT2))
    w2 = w2_ref[pl.ds(e, 1)][0]
    eo = jax.lax.dot_general(h.astype(jnp.bfloat16), w2,
                             (((1,), (1,)), ((), ())),
                             preferred_element_type=jnp.float32)
    eo = eo + b2_ref[pl.ds(e, 1)][0, 0]
    out_ref[...] = eo * g_ref[...]


def _moe_sparse(flat, flat_i, flat_p, exp_w1, exp_b1, exp_w2, exp_b2):
    ids = flat_i.reshape(-1)
    oneh = (ids[:, None] == jnp.arange(NEXP, dtype=ids.dtype)[None, :])
    incl = jnp.cumsum(oneh.astype(jnp.int32), axis=0)
    rank = jnp.take_along_axis(incl, ids[:, None], axis=1)[:, 0] - 1
    counts = incl[-1]
    padded = ((counts + _BA - 1) // _BA) * _BA
    off = jnp.concatenate([jnp.zeros((1,), jnp.int32),
                           jnp.cumsum(padded).astype(jnp.int32)])
    pos = off[ids] + rank
    tok = (jnp.arange(T * TOPK, dtype=jnp.int32) // TOPK)
    row_token = jnp.full((_PADT,), T, jnp.int32).at[pos].set(tok)
    row_gate = jnp.zeros((_PADT,), jnp.float32).at[pos].set(flat_p.reshape(-1))
    starts = jnp.arange(_NB, dtype=jnp.int32) * _BA
    block_expert = jnp.minimum(
        jnp.searchsorted(off[1:], starts, side='right'),
        NEXP - 1).astype(jnp.int32)

    z_pad = jnp.concatenate(
        [flat, jnp.zeros((_TPAD - T, E), jnp.float32)], axis=0)
    x_sorted = jnp.take(z_pad, row_token, axis=0).astype(jnp.bfloat16)

    grid_spec = pltpu.PrefetchScalarGridSpec(
        num_scalar_prefetch=1,
        grid=(_NB,),
        in_specs=[
            pl.BlockSpec((_BA, E), lambda i, be: (i, 0)),
            pl.BlockSpec((NEXP, HID, E), lambda i, be: (0, 0, 0)),
            pl.BlockSpec((NEXP, 1, HID), lambda i, be: (0, 0, 0)),
            pl.BlockSpec((NEXP, E, HID), lambda i, be: (0, 0, 0)),
            pl.BlockSpec((NEXP, 1, E), lambda i, be: (0, 0, 0)),
            pl.BlockSpec((_BA, 1), lambda i, be: (i, 0)),
        ],
        out_specs=pl.BlockSpec((_BA, E), lambda i, be: (i, 0)),
    )
    buf = pl.pallas_call(
        _moe_ffn_kernel,
        grid_spec=grid_spec,
        out_shape=jax.ShapeDtypeStruct((_PADT, E), jnp.float32),
    )(block_expert, x_sorted, exp_w1.astype(jnp.bfloat16),
      exp_b1.reshape(NEXP, 1, HID), exp_w2.astype(jnp.bfloat16),
      exp_b2.reshape(NEXP, 1, E), row_gate[:, None])
    pos2 = pos.reshape(T, TOPK)
    return jnp.take(buf, pos2[:, 0], axis=0) + jnp.take(buf, pos2[:, 1], axis=0)


_NPAD = 224
_DH = E // NH


def _attn_kernel(z_ref, ln_g_ref, ln_b_ref, win_ref, bin_ref, wout_ref,
                 bout_ref, out_ref):
    z = z_ref[0]
    m = jnp.mean(z, axis=-1, keepdims=True)
    v = jnp.mean((z - m) ** 2, axis=-1, keepdims=True)
    zn = (z - m) / jnp.sqrt(v + 1e-5) * ln_g_ref[0] + ln_b_ref[0]
    qkv = jax.lax.dot_general(zn.astype(jnp.bfloat16), win_ref[...],
                              (((1,), (1,)), ((), ())),
                              preferred_element_type=jnp.float32)
    qkv = qkv + bin_ref[0]
    kmask = (jax.lax.broadcasted_iota(jnp.int32, (_NPAD, _NPAD), 1)
             >= NPATCH)
    heads = []
    for h in range(NH):
        qh = qkv[:, h * _DH:(h + 1) * _DH]
        kh = qkv[:, E + h * _DH:E + (h + 1) * _DH]
        vh = qkv[:, 2 * E + h * _DH:2 * E + (h + 1) * _DH]
        s = jax.lax.dot_general(qh, kh, (((1,), (1,)), ((), ())),
                                preferred_element_type=jnp.float32)
        s = s * (1.0 / math.sqrt(_DH))
        s = jnp.where(kmask, -1e30, s)
        s = s - jnp.max(s, axis=-1, keepdims=True)
        p = jnp.exp(s)
        p = p / jnp.sum(p, axis=-1, keepdims=True)
        oh = jax.lax.dot_general(p.astype(jnp.bfloat16),
                                 vh.astype(jnp.bfloat16),
                                 (((1,), (0,)), ((), ())),
                                 preferred_element_type=jnp.float32)
        heads.append(oh)
    ao = jnp.concatenate(heads, axis=1)
    ao = jax.lax.dot_general(ao.astype(jnp.bfloat16), wout_ref[...],
                             (((1,), (1,)), ((), ())),
                             preferred_element_type=jnp.float32)
    out_ref[0] = z + ao + bout_ref[0]


def _attention(z, ln1_g, ln1_b, attn_in_w, attn_in_b, attn_out_w, attn_out_b):
    zp = jnp.pad(z, ((0, 0), (0, _NPAD - NPATCH), (0, 0)))
    out = pl.pallas_call(
        _attn_kernel,
        grid=(B,),
        in_specs=[
            pl.BlockSpec((1, _NPAD, E), lambda b: (b, 0, 0)),
            pl.BlockSpec((1, E), lambda b: (0, 0)),
            pl.BlockSpec((1, E), lambda b: (0, 0)),
            pl.BlockSpec((3 * E, E), lambda b: (0, 0)),
            pl.BlockSpec((1, 3 * E), lambda b: (0, 0)),
            pl.BlockSpec((E, E), lambda b: (0, 0)),
            pl.BlockSpec((1, E), lambda b: (0, 0)),
        ],
        out_specs=pl.BlockSpec((1, _NPAD, E), lambda b: (b, 0, 0)),
        out_shape=jax.ShapeDtypeStruct((B, _NPAD, E), jnp.float32),
    )(zp, ln1_g[None, :], ln1_b[None, :], attn_in_w.astype(jnp.bfloat16),
      attn_in_b[None, :], attn_out_w.astype(jnp.bfloat16), attn_out_b[None, :])
    return out[:, :NPATCH, :]


def _layernorm(x, g, b):
    m = jnp.mean(x, axis=-1, keepdims=True)
    v = jnp.mean((x - m) ** 2, axis=-1, keepdims=True)
    return (x - m) / jnp.sqrt(v + 1e-5) * g + b


def kernel(x, patch_w, patch_b, pos_embed, ln1_g, ln1_b, attn_in_w, attn_in_b,
           attn_out_w, attn_out_b, router_w, router_b, exp_w1, exp_b1, exp_w2,
           exp_b2, ln2_g, ln2_b, head_w, head_b):
    Bn = x.shape[0]
    hp = H // P
    patches = x.reshape(Bn, C, hp, P, hp, P).transpose(0, 2, 4, 1, 3, 5)
    patches = patches.reshape(Bn, hp * hp, C * P * P)
    z = patches @ patch_w.reshape(E, C * P * P).T + patch_b
    z = z + pos_embed

    z = _attention(z, ln1_g, ln1_b, attn_in_w, attn_in_b,
                   attn_out_w, attn_out_b)

    logits = z @ router_w.T + router_b
    probs = jax.nn.softmax(logits, axis=-1)
    topk_p, topk_i = jax.lax.top_k(probs, TOPK)
    flat = z.reshape(T, E)
    flat_i = topk_i.reshape(T, TOPK)
    flat_p = topk_p.reshape(T, TOPK)

    out = _moe_sparse(flat, flat_i, flat_p, exp_w1, exp_b1, exp_w2, exp_b2)

    z = out.reshape(Bn, -1, E)
    z = _layernorm(z, ln2_g, ln2_b)
    pooled = jnp.mean(z, axis=1)
    return pooled @ head_w.T + head_b
